# baseline (speedup 1.0000x reference)
import jax
import jax.numpy as jnp
from jax.experimental import pallas as pl
from jax.experimental.pallas import tpu as pltpu

TOKENS = 16384
HIDDEN = 2048
EXPERTS = 64
BLOCK_T = 1024
NBLOCKS = TOKENS // BLOCK_T
NBUF = 4


def _router_body(x_hbm, w_ref, b_ref, o_ref, xbuf, sems):
    w = w_ref[...].astype(jnp.bfloat16)
    bias = b_ref[...]

    def copy_in(block, slot):
        return pltpu.make_async_copy(
            x_hbm.at[pl.ds(block * BLOCK_T, BLOCK_T), :],
            xbuf.at[slot],
            sems.at[slot],
        )

    for slot in range(min(NBUF, NBLOCKS)):
        copy_in(slot, slot).start()

    for block in range(NBLOCKS):
        slot = block % NBUF
        copy_in(block, slot).wait()
        xb = xbuf[slot].astype(jnp.bfloat16)
        logits = jax.lax.dot_general(
            w, xb,
            dimension_numbers=(((1,), (1,)), ((), ())),
            preferred_element_type=jnp.float32,
        )
        o_ref[:, pl.ds(block * BLOCK_T, BLOCK_T)] = jnp.maximum(logits + bias, 0.0)
        nxt = block + NBUF
        if nxt < NBLOCKS:
            copy_in(nxt, slot).start()


@jax.jit
def kernel(x, W, b):
    b2 = b.reshape(EXPERTS, 1)
    out_t = pl.pallas_call(
        _router_body,
        in_specs=[
            pl.BlockSpec(memory_space=pltpu.MemorySpace.HBM),
            pl.BlockSpec(memory_space=pltpu.MemorySpace.VMEM),
            pl.BlockSpec(memory_space=pltpu.MemorySpace.VMEM),
        ],
        out_specs=pl.BlockSpec(memory_space=pltpu.MemorySpace.VMEM),
        out_shape=jax.ShapeDtypeStruct((EXPERTS, TOKENS), jnp.float32),
        scratch_shapes=[
            pltpu.VMEM((NBUF, BLOCK_T, HIDDEN), jnp.float32),
            pltpu.SemaphoreType.DMA((NBUF,)),
        ],
    )(x, W, b2)
    return out_t.T


# speedup vs baseline: 1.0681x; 1.0681x over previous
import jax
import jax.numpy as jnp
from jax.experimental import pallas as pl
from jax.experimental.pallas import tpu as pltpu

TOKENS = 16384
HIDDEN = 2048
EXPERTS = 64
BLOCK_T = 1024


def _router_body(x_ref, w_ref, b_ref, o_ref):
    logits = jax.lax.dot_general(
        w_ref[...], x_ref[...],
        dimension_numbers=(((1,), (1,)), ((), ())),
        preferred_element_type=jnp.float32,
    )
    o_ref[...] = jnp.maximum(logits + b_ref[...], 0.0)


@jax.jit
def kernel(x, W, b):
    b2 = b.reshape(EXPERTS, 1)
    grid = (TOKENS // BLOCK_T,)
    out_t = pl.pallas_call(
        _router_body,
        grid=grid,
        in_specs=[
            pl.BlockSpec((BLOCK_T, HIDDEN), lambda i: (i, 0)),
            pl.BlockSpec((EXPERTS, HIDDEN), lambda i: (0, 0)),
            pl.BlockSpec((EXPERTS, 1), lambda i: (0, 0)),
        ],
        out_specs=pl.BlockSpec((EXPERTS, BLOCK_T), lambda i: (0, i)),
        out_shape=jax.ShapeDtypeStruct((EXPERTS, TOKENS), jnp.float32),
        compiler_params=pltpu.CompilerParams(
            dimension_semantics=("parallel",),
        ),
    )(x, W, b2)
    return out_t.T


# speedup vs baseline: 1.0955x; 1.0257x over previous
import jax
import jax.numpy as jnp
from jax.experimental import pallas as pl
from jax.experimental.pallas import tpu as pltpu

TOKENS = 16384
HIDDEN = 2048
EXPERTS = 64
BLOCK_T = 1024


def _router_body(x_ref, w_ref, b_ref, o_ref):
    x = x_ref[...].astype(jnp.bfloat16)
    w = w_ref[...].astype(jnp.bfloat16)
    logits = jax.lax.dot_general(
        x, w,
        dimension_numbers=(((1,), (1,)), ((), ())),
        preferred_element_type=jnp.float32,
    )
    gated = jnp.maximum(logits + b_ref[...], 0.0)
    o_ref[...] = gated.T


@jax.jit
def kernel(x, W, b):
    b2 = b.reshape(1, EXPERTS)
    grid = (TOKENS // BLOCK_T,)
    out_t = pl.pallas_call(
        _router_body,
        grid=grid,
        in_specs=[
            pl.BlockSpec((BLOCK_T, HIDDEN), lambda i: (i, 0)),
            pl.BlockSpec((EXPERTS, HIDDEN), lambda i: (0, 0)),
            pl.BlockSpec((1, EXPERTS), lambda i: (0, 0)),
        ],
        out_specs=pl.BlockSpec((EXPERTS, BLOCK_T), lambda i: (0, i)),
        out_shape=jax.ShapeDtypeStruct((EXPERTS, TOKENS), jnp.float32),
        compiler_params=pltpu.CompilerParams(
            dimension_semantics=("parallel",),
        ),
    )(x, W, b2)
    return out_t.T
